# baseline (speedup 1.0000x reference)
import functools

import jax
import jax.numpy as jnp
from jax import lax
from jax.experimental import pallas as pl
from jax.experimental.pallas import tpu as pltpu
from jax.experimental.pallas import tpu_sc as plsc

BATCH = 4
SEQ_LEN = 4096
D_MODEL = 2048
NUM_BUCKETS = 64
MAX_SKIP_LAYERS = 12
_MAX_SKIPPED_FLOPS = float(MAX_SKIP_LAYERS * 12 * D_MODEL * D_MODEL * SEQ_LEN)

_N = BATCH * SEQ_LEN
_NC = 2
_NS = 16
_NW = _NC * _NS
_TPW = _N // _NW
_C = 8
_NB = 4
_NCHUNK = _TPW // _C
_NGRP = _NCHUNK // _NB


def _sc_body(x_hbm, csf_hbm, tab_hbm, out_hbm,
             csf_v, tab_sh, eb0, eb1, xb0, xb1, xb2, xb3,
             es0, es1, ls0, ls1, ls2, ls3, ss0, ss1, ss2, ss3):
    c = lax.axis_index("c")
    s = lax.axis_index("s")
    wid = s * _NC + c
    base = wid * _TPW
    ebs = (eb0, eb1)
    esems = (es0, es1)
    xbs = (xb0, xb1, xb2, xb3)
    lsems = (ls0, ls1, ls2, ls3)
    ssems = (ss0, ss1, ss2, ss3)

    @pl.when(s == 0)
    def _():
        pltpu.sync_copy(tab_hbm, tab_sh)

    pltpu.sync_copy(csf_hbm.at[pl.ds(base, _TPW)], csf_v.at[pl.ds(0, _TPW)])
    plsc.subcore_barrier()

    def ld(ci, b):
        return pltpu.async_copy(
            x_hbm.at[pl.ds(base + ci * _C, _C)], xbs[b], lsems[b])

    def st(ci, b):
        return pltpu.async_copy(
            xbs[b], out_hbm.at[pl.ds(base + ci * _C, _C)], ssems[b])

    def chunk_idx(cj):
        f = csf_v[pl.ds(cj * _C, 16)]
        frac = f / jnp.float32(_MAX_SKIPPED_FLOPS)
        i = (frac * jnp.float32(NUM_BUCKETS)).astype(jnp.int32)
        return jnp.clip(i, 0, NUM_BUCKETS - 1)

    def fire_rows(cj, par):
        idxv = chunk_idx(cj)
        for t in range(_C):
            pltpu.async_copy(tab_sh.at[idxv[t]], ebs[par].at[t], esems[par])

    ld(0, 0)
    ld(1, 1)
    fire_rows(0, 0)

    def grp(g, carry):
        for b in range(_NB):
            ci = g * _NB + b
            par = b % 2
            pltpu.make_async_copy(x_hbm.at[pl.ds(0, _C)], xbs[b], lsems[b]).wait()
            pltpu.make_async_copy(tab_sh.at[pl.ds(0, _C)], ebs[par], esems[par]).wait()

            @pl.when(ci + 1 < _NCHUNK)
            def _(ci=ci, par=par):
                fire_rows(ci + 1, 1 - par)

            for t in range(_C):

                def jbody(j, c2, t=t, b=b, par=par):
                    for k in range(8):
                        sl = pl.ds(j * 128 + k * 16, 16)
                        xbs[b][t, sl] = xbs[b][t, sl] + ebs[par][t, sl]
                    return c2

                lax.fori_loop(0, D_MODEL // 128, jbody, 0)
            st(ci, b)
            b2 = (b + 2) % _NB

            @pl.when(jnp.logical_and(ci + 2 < _NCHUNK, ci >= 2))
            def _(b2=b2, ci=ci):
                pltpu.make_async_copy(
                    xbs[b2], out_hbm.at[pl.ds(0, _C)], ssems[b2]).wait()
                ld(ci + 2, b2)

            @pl.when(jnp.logical_and(ci + 2 < _NCHUNK, ci < 2))
            def _(b2=b2, ci=ci):
                ld(ci + 2, b2)

        return carry

    lax.fori_loop(0, _NGRP, grp, 0)

    for b in range(_NB):
        pltpu.make_async_copy(
            xbs[b], out_hbm.at[pl.ds(0, _C)], ssems[b]).wait()


@jax.jit
def _sc_call(x2, csf1, tab):
    mesh = plsc.VectorSubcoreMesh(core_axis_name="c", subcore_axis_name="s")
    f = functools.partial(
        pl.kernel,
        out_type=jax.ShapeDtypeStruct((_N, D_MODEL), jnp.float32),
        mesh=mesh,
        scratch_types=[
            pltpu.VMEM((_TPW + 16,), jnp.float32),
            pltpu.VMEM_SHARED((NUM_BUCKETS, D_MODEL), jnp.float32),
            pltpu.VMEM((_C, D_MODEL), jnp.float32),
            pltpu.VMEM((_C, D_MODEL), jnp.float32),
        ] + [pltpu.VMEM((_C, D_MODEL), jnp.float32)] * _NB
          + [pltpu.SemaphoreType.DMA] * (2 * _NB + 2),
    )(_sc_body)
    return f(x2, csf1, tab)


def kernel(x, cumulative_skipped_flops, step_embeddings_weight):
    x2 = x.reshape(_N, D_MODEL)
    csf1 = cumulative_skipped_flops.reshape(_N)
    out = _sc_call(x2, csf1, step_embeddings_weight)
    return out.reshape(BATCH, SEQ_LEN, D_MODEL)


# speedup vs baseline: 1.0647x; 1.0647x over previous
import functools

import jax
import jax.numpy as jnp
from jax import lax
from jax.experimental import pallas as pl
from jax.experimental.pallas import tpu as pltpu
from jax.experimental.pallas import tpu_sc as plsc

BATCH = 4
SEQ_LEN = 4096
D_MODEL = 2048
NUM_BUCKETS = 64
MAX_SKIP_LAYERS = 12
_MAX_SKIPPED_FLOPS = float(MAX_SKIP_LAYERS * 12 * D_MODEL * D_MODEL * SEQ_LEN)

_N = BATCH * SEQ_LEN
_NC = 2
_NS = 16
_NW = _NC * _NS
_TPW = _N // _NW
_C = 8
_NB = 4
_NCHUNK = _TPW // _C
_NGRP = _NCHUNK // _NB


def _sc_body(x_hbm, csf_hbm, tab_hbm, out_hbm,
             csf_v, tab_sh, eb0, eb1, xb0, xb1, xb2, xb3,
             es0, es1, ls0, ls1, ls2, ls3, ss0, ss1, ss2, ss3):
    c = lax.axis_index("c")
    s = lax.axis_index("s")
    wid = s * _NC + c
    base = wid * _TPW
    ebs = (eb0, eb1)
    esems = (es0, es1)
    xbs = (xb0, xb1, xb2, xb3)
    lsems = (ls0, ls1, ls2, ls3)
    ssems = (ss0, ss1, ss2, ss3)

    @pl.when(s == 0)
    def _():
        pltpu.sync_copy(tab_hbm, tab_sh)

    pltpu.sync_copy(csf_hbm.at[pl.ds(base, _TPW)], csf_v.at[pl.ds(0, _TPW)])
    plsc.subcore_barrier()

    def ld(ci, b):
        return pltpu.async_copy(
            x_hbm.at[pl.ds(base + ci * _C, _C)], xbs[b], lsems[b])

    def st(ci, b):
        return pltpu.async_copy(
            xbs[b], out_hbm.at[pl.ds(base + ci * _C, _C)], ssems[b])

    def chunk_idx(cj):
        f = csf_v[pl.ds(cj * _C, 16)]
        frac = f / jnp.float32(_MAX_SKIPPED_FLOPS)
        i = (frac * jnp.float32(NUM_BUCKETS)).astype(jnp.int32)
        return jnp.clip(i, 0, NUM_BUCKETS - 1)

    def fire_rows(cj, par):
        idxv = chunk_idx(cj)
        for t in range(_C):
            pltpu.async_copy(tab_sh.at[idxv[t]], ebs[par].at[t], esems[par])

    ld(0, 0)
    ld(1, 1)
    fire_rows(0, 0)

    def grp(g, carry):
        for b in range(_NB):
            ci = g * _NB + b
            par = b % 2
            pltpu.make_async_copy(x_hbm.at[pl.ds(0, _C)], xbs[b], lsems[b]).wait()
            pltpu.make_async_copy(tab_sh.at[pl.ds(0, _C)], ebs[par], esems[par]).wait()

            @pl.when(ci + 1 < _NCHUNK)
            def _(ci=ci, par=par):
                fire_rows(ci + 1, 1 - par)

            b2 = (b + 2) % _NB

            @pl.when(jnp.logical_and(ci + 2 < _NCHUNK, ci >= 2))
            def _(b2=b2, ci=ci):
                pltpu.make_async_copy(
                    xbs[b2], out_hbm.at[pl.ds(0, _C)], ssems[b2]).wait()
                ld(ci + 2, b2)

            @pl.when(jnp.logical_and(ci + 2 < _NCHUNK, ci < 2))
            def _(b2=b2, ci=ci):
                ld(ci + 2, b2)

            for t in range(_C):

                def jbody(j, c2, t=t, b=b, par=par):
                    for k in range(8):
                        sl = pl.ds(j * 128 + k * 16, 16)
                        plsc.addupdate(xbs[b].at[t, sl], ebs[par][t, sl])
                    return c2

                lax.fori_loop(0, D_MODEL // 128, jbody, 0)
            st(ci, b)

        return carry

    lax.fori_loop(0, _NGRP, grp, 0)

    for b in range(_NB):
        pltpu.make_async_copy(
            xbs[b], out_hbm.at[pl.ds(0, _C)], ssems[b]).wait()


@jax.jit
def _sc_call(x2, csf1, tab):
    mesh = plsc.VectorSubcoreMesh(core_axis_name="c", subcore_axis_name="s")
    f = functools.partial(
        pl.kernel,
        out_type=jax.ShapeDtypeStruct((_N, D_MODEL), jnp.float32),
        mesh=mesh,
        scratch_types=[
            pltpu.VMEM((_TPW + 16,), jnp.float32),
            pltpu.VMEM_SHARED((NUM_BUCKETS, D_MODEL), jnp.float32),
            pltpu.VMEM((_C, D_MODEL), jnp.float32),
            pltpu.VMEM((_C, D_MODEL), jnp.float32),
        ] + [pltpu.VMEM((_C, D_MODEL), jnp.float32)] * _NB
          + [pltpu.SemaphoreType.DMA] * (2 * _NB + 2),
    )(_sc_body)
    return f(x2, csf1, tab)


def kernel(x, cumulative_skipped_flops, step_embeddings_weight):
    x2 = x.reshape(_N, D_MODEL)
    csf1 = cumulative_skipped_flops.reshape(_N)
    out = _sc_call(x2, csf1, step_embeddings_weight)
    return out.reshape(BATCH, SEQ_LEN, D_MODEL)
